# Initial kernel scaffold; baseline (speedup 1.0000x reference)
import jax
import jax.numpy as jnp
from jax.experimental import pallas as pl


def kernel(x, a, W1, b1, gamma, beta, Wp1, bp1, Wp2, bp2):
    raise NotImplementedError("write your pallas kernel here")


# speedup vs baseline: 1.0492x; 1.0492x over previous
import functools

import jax
import jax.numpy as jnp
from jax.experimental import pallas as pl
from jax.experimental.pallas import tpu as pltpu

N = 10000
BR = 400


def _entry_body(x_ref, w_ref, b_ref, o_ref):
    z = jnp.dot(x_ref[...], w_ref[...], preferred_element_type=jnp.float32)
    o_ref[...] = jnp.maximum(z + b_ref[...], 0.0)


def _prop_body(a_ref, v_ref, o_ref):
    o_ref[...] = jnp.dot(a_ref[...], v_ref[...],
                         preferred_element_type=jnp.float32)


def _head_body(z_ref, g_ref, be_ref, wp1_ref, bp1_ref, wp2_ref, bp2_ref,
               zn_ref, p_ref):
    z = z_ref[...]
    mean = jnp.mean(z, axis=0, keepdims=True)
    var = jnp.mean(jnp.square(z - mean), axis=0, keepdims=True)
    zn = (z - mean) / jnp.sqrt(var + 1e-5) * g_ref[...] + be_ref[...]
    zn_ref[...] = zn
    h = jnp.maximum(
        jnp.dot(zn, wp1_ref[...], preferred_element_type=jnp.float32)
        + bp1_ref[...], 0.0)
    p_ref[...] = (jnp.dot(h, wp2_ref[...], preferred_element_type=jnp.float32)
                  + bp2_ref[...])


@functools.partial(jax.jit, static_argnums=())
def kernel(x, a, W1, b1, gamma, beta, Wp1, bp1, Wp2, bp2):
    emb = W1.shape[1]
    proj = Wp2.shape[1]
    b1r = b1.reshape(1, emb)
    gr = gamma.reshape(1, emb)
    ber = beta.reshape(1, emb)
    bp1r = bp1.reshape(1, proj)
    bp2r = bp2.reshape(1, proj)

    z1 = pl.pallas_call(
        _entry_body,
        out_shape=jax.ShapeDtypeStruct((N, emb), jnp.float32),
    )(x, W1, b1r)

    nb = N // BR
    prop = pl.pallas_call(
        _prop_body,
        grid=(nb,),
        in_specs=[
            pl.BlockSpec((BR, N), lambda i: (i, 0)),
            pl.BlockSpec((N, emb), lambda i: (0, 0)),
        ],
        out_specs=pl.BlockSpec((BR, emb), lambda i: (i, 0)),
        out_shape=jax.ShapeDtypeStruct((N, emb), jnp.float32),
        compiler_params=pltpu.CompilerParams(
            dimension_semantics=("arbitrary",)),
    )
    z2 = prop(a, z1)
    z3 = prop(a, z2)

    zn, p = pl.pallas_call(
        _head_body,
        out_shape=(
            jax.ShapeDtypeStruct((N, emb), jnp.float32),
            jax.ShapeDtypeStruct((N, proj), jnp.float32),
        ),
    )(z3, gr, ber, Wp1, bp1r, Wp2, bp2r)
    return (zn, p)


# speedup vs baseline: 1.0523x; 1.0030x over previous
import functools

import jax
import jax.numpy as jnp
from jax.experimental import pallas as pl
from jax.experimental.pallas import tpu as pltpu

N = 10000
BR = 200


def _entry_body(x_ref, w_ref, b_ref, o_ref):
    z = jnp.dot(x_ref[...], w_ref[...], preferred_element_type=jnp.float32)
    o_ref[...] = jnp.maximum(z + b_ref[...], 0.0)


def _prop_body(a_ref, v_ref, o_ref):
    o_ref[...] = jnp.dot(a_ref[...], v_ref[...],
                         preferred_element_type=jnp.float32)


def _head_body(z_ref, g_ref, be_ref, wp1_ref, bp1_ref, wp2_ref, bp2_ref,
               zn_ref, p_ref):
    z = z_ref[...]
    mean = jnp.mean(z, axis=0, keepdims=True)
    var = jnp.mean(jnp.square(z - mean), axis=0, keepdims=True)
    zn = (z - mean) / jnp.sqrt(var + 1e-5) * g_ref[...] + be_ref[...]
    zn_ref[...] = zn
    h = jnp.maximum(
        jnp.dot(zn, wp1_ref[...], preferred_element_type=jnp.float32)
        + bp1_ref[...], 0.0)
    p_ref[...] = (jnp.dot(h, wp2_ref[...], preferred_element_type=jnp.float32)
                  + bp2_ref[...])


@functools.partial(jax.jit, static_argnums=())
def kernel(x, a, W1, b1, gamma, beta, Wp1, bp1, Wp2, bp2):
    emb = W1.shape[1]
    proj = Wp2.shape[1]
    b1r = b1.reshape(1, emb)
    gr = gamma.reshape(1, emb)
    ber = beta.reshape(1, emb)
    bp1r = bp1.reshape(1, proj)
    bp2r = bp2.reshape(1, proj)

    z1 = pl.pallas_call(
        _entry_body,
        out_shape=jax.ShapeDtypeStruct((N, emb), jnp.float32),
    )(x, W1, b1r)

    nb = N // BR
    prop = pl.pallas_call(
        _prop_body,
        grid=(nb,),
        in_specs=[
            pl.BlockSpec((BR, N), lambda i: (i, 0)),
            pl.BlockSpec((N, emb), lambda i: (0, 0)),
        ],
        out_specs=pl.BlockSpec((BR, emb), lambda i: (i, 0)),
        out_shape=jax.ShapeDtypeStruct((N, emb), jnp.float32),
        compiler_params=pltpu.CompilerParams(
            dimension_semantics=("arbitrary",)),
    )
    z2 = prop(a, z1)
    z3 = prop(a, z2)

    zn, p = pl.pallas_call(
        _head_body,
        out_shape=(
            jax.ShapeDtypeStruct((N, emb), jnp.float32),
            jax.ShapeDtypeStruct((N, proj), jnp.float32),
        ),
    )(z3, gr, ber, Wp1, bp1r, Wp2, bp2r)
    return (zn, p)


# speedup vs baseline: 1.1074x; 1.0524x over previous
import jax
import jax.numpy as jnp
from jax.experimental import pallas as pl
from jax.experimental.pallas import tpu as pltpu

N = 10000
BR = 200
NB = N // BR


def _fused_body(x_ref, a_ref, w1_ref, b1_ref, g_ref, be_ref, wp1_ref,
                bp1_ref, wp2_ref, bp2_ref, zn_ref, p_ref,
                z1_s, z2_s, z3_s):
    i = pl.program_id(0)

    @pl.when(i == 0)
    def _entry():
        z = jnp.dot(x_ref[...], w1_ref[...],
                    preferred_element_type=jnp.float32)
        z1_s[...] = jnp.maximum(z + b1_ref[...], 0.0)

    @pl.when(i < NB)
    def _prop1():
        z2_s[pl.ds(i * BR, BR), :] = jnp.dot(
            a_ref[...], z1_s[...], preferred_element_type=jnp.float32)

    @pl.when((i >= NB) & (i < 2 * NB))
    def _prop2():
        j = i - NB
        z3_s[pl.ds(j * BR, BR), :] = jnp.dot(
            a_ref[...], z2_s[...], preferred_element_type=jnp.float32)

    @pl.when(i == 2 * NB)
    def _head():
        z = z3_s[...]
        mean = jnp.mean(z, axis=0, keepdims=True)
        var = jnp.mean(jnp.square(z - mean), axis=0, keepdims=True)
        zn = (z - mean) / jnp.sqrt(var + 1e-5) * g_ref[...] + be_ref[...]
        zn_ref[...] = zn
        h = jnp.maximum(
            jnp.dot(zn, wp1_ref[...], preferred_element_type=jnp.float32)
            + bp1_ref[...], 0.0)
        p_ref[...] = (
            jnp.dot(h, wp2_ref[...], preferred_element_type=jnp.float32)
            + bp2_ref[...])


def _a_index(i):
    blk = jnp.where(i < NB, i, jnp.where(i < 2 * NB, i - NB, NB - 1))
    return (blk, 0)


def kernel(x, a, W1, b1, gamma, beta, Wp1, bp1, Wp2, bp2):
    emb = W1.shape[1]
    proj = Wp2.shape[1]
    b1r = b1.reshape(1, emb)
    gr = gamma.reshape(1, emb)
    ber = beta.reshape(1, emb)
    bp1r = bp1.reshape(1, proj)
    bp2r = bp2.reshape(1, proj)

    zn, p = pl.pallas_call(
        _fused_body,
        grid=(2 * NB + 1,),
        in_specs=[
            pl.BlockSpec((N, x.shape[1]), lambda i: (0, 0)),
            pl.BlockSpec((BR, N), _a_index),
            pl.BlockSpec((x.shape[1], emb), lambda i: (0, 0)),
            pl.BlockSpec((1, emb), lambda i: (0, 0)),
            pl.BlockSpec((1, emb), lambda i: (0, 0)),
            pl.BlockSpec((1, emb), lambda i: (0, 0)),
            pl.BlockSpec((emb, proj), lambda i: (0, 0)),
            pl.BlockSpec((1, proj), lambda i: (0, 0)),
            pl.BlockSpec((proj, proj), lambda i: (0, 0)),
            pl.BlockSpec((1, proj), lambda i: (0, 0)),
        ],
        out_specs=(
            pl.BlockSpec((N, emb), lambda i: (0, 0)),
            pl.BlockSpec((N, proj), lambda i: (0, 0)),
        ),
        out_shape=(
            jax.ShapeDtypeStruct((N, emb), jnp.float32),
            jax.ShapeDtypeStruct((N, proj), jnp.float32),
        ),
        scratch_shapes=[
            pltpu.VMEM((N, emb), jnp.float32),
            pltpu.VMEM((N, emb), jnp.float32),
            pltpu.VMEM((N, emb), jnp.float32),
        ],
        compiler_params=pltpu.CompilerParams(
            dimension_semantics=("arbitrary",)),
    )(x, a, W1, b1r, gr, ber, Wp1, bp1r, Wp2, bp2r)
    return (zn, p)
